# baseline (speedup 1.0000x reference)
import functools

import jax
import jax.numpy as jnp
from jax import lax
from jax.experimental import pallas as pl
from jax.experimental.pallas import tpu as pltpu
from jax.experimental.pallas import tpu_sc as plsc

N_USER = 1000000
N_ITEM = 100000
N_INP = 64
BATCH = 16384

_info = plsc.get_sparse_core_info()
_NC, _NS = _info.num_cores, _info.num_subcores
_NW = _NC * _NS

_NG_U = N_USER // 128
_NG_I = N_ITEM // 128
_TAIL_U = N_USER - _NG_U * 128
_TAIL_I = N_ITEM - _NG_I * 128
_QU, _RU = divmod(_NG_U, _NW)
_QI, _RI = divmod(_NG_I, _NW)
_MAXCH_U = _QU + 1
_MAXCH_I = _QI + 1
_NVEC = BATCH // 16


def _gather_body(tab_u, tab_i, nid_u_hbm, nid_i_hbm,
                 out_u_hbm, out_i_hbm,
                 ids_v, hits_v, buf0, buf1, buf2, stage_v,
                 sem0, sem1, sem2, semw):
    wid = lax.axis_index("s") * _NC + lax.axis_index("c")
    iota = lax.iota(jnp.int32, 16)

    def process_table(tab, nid, out, q, r, tail_group, tail_lane, tail_wid,
                      maxch):
        s_w = q * wid + jnp.minimum(wid, r)
        nch = q + jnp.where(wid < r, 1, 0)

        def fetch(t, buf, sem):
            @pl.when(t < nch)
            def _():
                lane = pl.multiple_of((s_w + t) * 128, 128)
                pltpu.make_async_copy(tab.at[:, pl.ds(lane, 128)],
                                      buf, sem).start()

        fetch(0, buf0, sem0)
        fetch(1, buf1, sem1)
        pltpu.sync_copy(nid, ids_v)

        def scan(k, off):
            vec = ids_v[pl.ds(k * 16, 16)]
            g = lax.shift_right_logical(vec, 7)
            m = (g >= s_w) & (g < s_w + nch)
            cnt = jnp.sum(jnp.where(m, 1, 0))

            def put(h, carry):
                o, mm = carry
                p = jnp.min(jnp.where(mm, iota, 99))
                hits_v[pl.ds(o, 16)] = jnp.full((16,), k * 16, jnp.int32) + p
                return (o + 1, mm & (iota != p))
            off2, _ = lax.fori_loop(0, cnt, put, (off, m))
            return off2
        nhits = lax.fori_loop(0, _NVEC, scan, 0)
        nhvec = (nhits + 15) // 16

        def drain_writes(n):
            def w1(i, c):
                pltpu.make_async_copy(stage_v.at[0], out.at[0], semw).wait()
                return c
            lax.fori_loop(0, n, w1, 0)

        def emit_row(buf, lane, j, rc):
            drain_writes(jnp.where(rc >= 16, 1, 0))
            slot = rc & 15
            lvec = jnp.full((16,), lane, jnp.int32)
            for c0 in range(0, N_INP, 16):
                stage_v[slot, pl.ds(c0, 16)] = plsc.load_gather(
                    buf, [c0 + iota, lvec])
            pltpu.make_async_copy(stage_v.at[slot], out.at[j], semw).start()
            return rc + 1

        def select_hits(cg, buf, rc):
            def hscan(k, rc2):
                jv = hits_v[pl.ds(k * 16, 16)]
                valid = (k * 16 + iota) < nhits
                idv = plsc.load_gather(ids_v, [jv & (BATCH - 1)])
                m = valid & (lax.shift_right_logical(idv, 7) == cg)
                cnt = jnp.sum(jnp.where(m, 1, 0))

                def put(h, carry):
                    rc3, mm = carry
                    p = jnp.min(jnp.where(mm, iota, 99))
                    j = jnp.sum(jnp.where(iota == p, jv, 0))
                    idsel = jnp.sum(jnp.where(iota == p, idv, 0))
                    rc4 = emit_row(buf, idsel - cg * 128, j, rc3)
                    return (rc4, mm & (iota != p))
                rc5, _ = lax.fori_loop(0, cnt, put, (rc2, m))
                return rc5
            return lax.fori_loop(0, nhvec, hscan, rc)

        def triple(t3, rc):
            for b in range(3):
                t = 3 * t3 + b
                buf = (buf0, buf1, buf2)[b]
                sem = (sem0, sem1, sem2)[b]
                nbuf = (buf0, buf1, buf2)[(b + 2) % 3]
                nsem = (sem0, sem1, sem2)[(b + 2) % 3]
                live = t < nch

                @pl.when(live)
                def _():
                    pltpu.make_async_copy(tab.at[:, pl.ds(0, 128)],
                                          buf, sem).wait()
                cg_eff = jnp.where(live, s_w + t, -1)
                rc = select_hits(cg_eff, buf, rc)

                @pl.when(t + 2 < nch)
                def _():
                    lane = pl.multiple_of((s_w + t + 2) * 128, 128)
                    pltpu.make_async_copy(tab.at[:, pl.ds(lane, 128)],
                                          nbuf, nsem).start()
            return rc
        rc = lax.fori_loop(0, (maxch + 2) // 3, triple, 0)
        drain_writes(jnp.minimum(rc, 16))

        @pl.when(wid == _NW - 1)
        def _():
            for f in range(N_INP):
                pltpu.make_async_copy(
                    tab.at[f, pl.ds(tail_group * 128, tail_wid)],
                    buf0.at[f, pl.ds(0, tail_wid)], sem0).start()
            for f in range(N_INP):
                pltpu.make_async_copy(
                    tab.at[f, pl.ds(tail_group * 128, tail_wid)],
                    buf0.at[f, pl.ds(0, tail_wid)], sem0).wait()

            def tscan(k, rc2):
                vec = ids_v[pl.ds(k * 16, 16)]
                m = vec >= tail_lane
                cnt = jnp.sum(jnp.where(m, 1, 0))

                def put(h, carry):
                    rc3, mm = carry
                    p = jnp.min(jnp.where(mm, iota, 99))
                    j = k * 16 + p
                    idsel = jnp.sum(jnp.where(iota == p, vec, 0))
                    rc4 = emit_row(buf0, idsel - tail_lane, j, rc3)
                    return (rc4, mm & (iota != p))
                rc5, _ = lax.fori_loop(0, cnt, put, (rc2, m))
                return rc5
            rct = lax.fori_loop(0, _NVEC, tscan, 0)
            drain_writes(jnp.minimum(rct, 16))

    process_table(tab_i, nid_i_hbm, out_i_hbm, _QI, _RI, _NG_I,
                  _NG_I * 128, _TAIL_I, _MAXCH_I)
    process_table(tab_u, nid_u_hbm, out_u_hbm, _QU, _RU, _NG_U,
                  _NG_U * 128, _TAIL_U, _MAXCH_U)


@jax.jit
def kernel(embed_user, embed_item, nid_user, nid_item):
    tab_u = embed_user.T
    tab_i = embed_item.T
    mesh = plsc.VectorSubcoreMesh(core_axis_name="c", subcore_axis_name="s")
    run = functools.partial(
        pl.kernel,
        mesh=mesh,
        out_type=(
            jax.ShapeDtypeStruct((BATCH, N_INP), jnp.float32),
            jax.ShapeDtypeStruct((BATCH, N_INP), jnp.float32),
        ),
        scratch_types=[
            pltpu.VMEM((BATCH,), jnp.int32),
            pltpu.VMEM((BATCH + 16,), jnp.int32),
            pltpu.VMEM((N_INP, 128), jnp.float32),
            pltpu.VMEM((N_INP, 128), jnp.float32),
            pltpu.VMEM((N_INP, 128), jnp.float32),
            pltpu.VMEM((16, N_INP), jnp.float32),
            pltpu.SemaphoreType.DMA,
            pltpu.SemaphoreType.DMA,
            pltpu.SemaphoreType.DMA,
            pltpu.SemaphoreType.DMA,
        ],
        compiler_params=pltpu.CompilerParams(needs_layout_passes=False),
    )(_gather_body)
    return run(tab_u, tab_i, nid_user, nid_item)


# speedup vs baseline: 1.1545x; 1.1545x over previous
import functools

import jax
import jax.numpy as jnp
from jax import lax
from jax.experimental import pallas as pl
from jax.experimental.pallas import tpu as pltpu
from jax.experimental.pallas import tpu_sc as plsc

N_USER = 1000000
N_ITEM = 100000
N_INP = 64
BATCH = 16384

_info = plsc.get_sparse_core_info()
_NC, _NS = _info.num_cores, _info.num_subcores
_NW = _NC * _NS

_NG_U = N_USER // 128
_NG_I = N_ITEM // 128
_TAIL_U = N_USER - _NG_U * 128
_TAIL_I = N_ITEM - _NG_I * 128
_QU, _RU = divmod(_NG_U, _NW)
_QI, _RI = divmod(_NG_I, _NW)
_MAXCH_U = _QU + 1
_MAXCH_I = _QI + 1
_NVEC = BATCH // 16


def _gather_body(tab_u, tab_i, nid_u_hbm, nid_i_hbm,
                 out_u_hbm, out_i_hbm,
                 ids_v, hits_v, buf0, buf1, buf2, stage_v,
                 sem0, sem1, sem2, semw):
    wid = lax.axis_index("s") * _NC + lax.axis_index("c")
    iota = lax.iota(jnp.int32, 16)

    def process_table(tab, nid, out, q, r, tail_group, tail_lane, tail_wid,
                      maxch):
        s_w = q * wid + jnp.minimum(wid, r)
        nch = q + jnp.where(wid < r, 1, 0)

        def fetch(t, buf, sem):
            @pl.when(t < nch)
            def _():
                lane = pl.multiple_of((s_w + t) * 128, 128)
                pltpu.make_async_copy(tab.at[:, pl.ds(lane, 128)],
                                      buf, sem).start()

        fetch(0, buf0, sem0)
        fetch(1, buf1, sem1)
        pltpu.sync_copy(nid, ids_v)

        def scan(k, off):
            vec = ids_v[pl.ds(k * 16, 16)]
            g = lax.shift_right_logical(vec, 7)
            m = (g >= s_w) & (g < s_w + nch)
            cnt = plsc.all_reduce_population_count(m)[0]

            def put(h, carry):
                o, mm = carry
                pv = plsc.all_reduce_ffs(mm)
                hits_v[pl.ds(o, 16)] = k * 16 + pv
                return (o + 1, mm & (iota != pv))
            off2, _ = lax.fori_loop(0, cnt, put, (off, m))
            return off2
        nhits = lax.fori_loop(0, _NVEC, scan, 0)
        nhvec = (nhits + 15) // 16

        def drain_writes(n):
            def w1(i, c):
                pltpu.make_async_copy(stage_v.at[0], out.at[0], semw).wait()
                return c
            lax.fori_loop(0, n, w1, 0)

        def emit_row(buf, lane, j, rc):
            drain_writes(jnp.where(rc >= 16, 1, 0))
            slot = rc & 15
            lvec = jnp.full((16,), lane, jnp.int32)
            for c0 in range(0, N_INP, 16):
                stage_v[slot, pl.ds(c0, 16)] = plsc.load_gather(
                    buf, [c0 + iota, lvec])
            pltpu.make_async_copy(stage_v.at[slot], out.at[j], semw).start()
            return rc + 1

        def select_hits(cg, buf, rc):
            def hscan(k, rc2):
                jv = hits_v[pl.ds(k * 16, 16)]
                valid = (k * 16 + iota) < nhits
                idv = plsc.load_gather(ids_v, [jv & (BATCH - 1)])
                m = valid & (lax.shift_right_logical(idv, 7) == cg)
                cnt = plsc.all_reduce_population_count(m)[0]

                def put(h, carry):
                    rc3, mm = carry
                    pv = plsc.all_reduce_ffs(mm)
                    j = jv.at[pv].get(mode="promise_in_bounds")[0]
                    idsel = idv.at[pv].get(mode="promise_in_bounds")[0]
                    rc4 = emit_row(buf, idsel - cg * 128, j, rc3)
                    return (rc4, mm & (iota != pv))
                rc5, _ = lax.fori_loop(0, cnt, put, (rc2, m))
                return rc5
            return lax.fori_loop(0, nhvec, hscan, rc)

        def triple(t3, rc):
            for b in range(3):
                t = 3 * t3 + b
                buf = (buf0, buf1, buf2)[b]
                sem = (sem0, sem1, sem2)[b]
                nbuf = (buf0, buf1, buf2)[(b + 2) % 3]
                nsem = (sem0, sem1, sem2)[(b + 2) % 3]
                live = t < nch

                @pl.when(live)
                def _():
                    pltpu.make_async_copy(tab.at[:, pl.ds(0, 128)],
                                          buf, sem).wait()
                cg_eff = jnp.where(live, s_w + t, -1)
                rc = select_hits(cg_eff, buf, rc)

                @pl.when(t + 2 < nch)
                def _():
                    lane = pl.multiple_of((s_w + t + 2) * 128, 128)
                    pltpu.make_async_copy(tab.at[:, pl.ds(lane, 128)],
                                          nbuf, nsem).start()
            return rc
        rc = lax.fori_loop(0, (maxch + 2) // 3, triple, 0)
        drain_writes(jnp.minimum(rc, 16))

        @pl.when(wid == _NW - 1)
        def _():
            for f in range(N_INP):
                pltpu.make_async_copy(
                    tab.at[f, pl.ds(tail_group * 128, tail_wid)],
                    buf0.at[f, pl.ds(0, tail_wid)], sem0).start()
            for f in range(N_INP):
                pltpu.make_async_copy(
                    tab.at[f, pl.ds(tail_group * 128, tail_wid)],
                    buf0.at[f, pl.ds(0, tail_wid)], sem0).wait()

            def tscan(k, rc2):
                vec = ids_v[pl.ds(k * 16, 16)]
                m = vec >= tail_lane
                cnt = plsc.all_reduce_population_count(m)[0]

                def put(h, carry):
                    rc3, mm = carry
                    pv = plsc.all_reduce_ffs(mm)
                    j = (k * 16 + pv)[0]
                    idsel = vec.at[pv].get(mode="promise_in_bounds")[0]
                    rc4 = emit_row(buf0, idsel - tail_lane, j, rc3)
                    return (rc4, mm & (iota != pv))
                rc5, _ = lax.fori_loop(0, cnt, put, (rc2, m))
                return rc5
            rct = lax.fori_loop(0, _NVEC, tscan, 0)
            drain_writes(jnp.minimum(rct, 16))

    process_table(tab_i, nid_i_hbm, out_i_hbm, _QI, _RI, _NG_I,
                  _NG_I * 128, _TAIL_I, _MAXCH_I)
    process_table(tab_u, nid_u_hbm, out_u_hbm, _QU, _RU, _NG_U,
                  _NG_U * 128, _TAIL_U, _MAXCH_U)


@jax.jit
def kernel(embed_user, embed_item, nid_user, nid_item):
    tab_u = embed_user.T
    tab_i = embed_item.T
    mesh = plsc.VectorSubcoreMesh(core_axis_name="c", subcore_axis_name="s")
    run = functools.partial(
        pl.kernel,
        mesh=mesh,
        out_type=(
            jax.ShapeDtypeStruct((BATCH, N_INP), jnp.float32),
            jax.ShapeDtypeStruct((BATCH, N_INP), jnp.float32),
        ),
        scratch_types=[
            pltpu.VMEM((BATCH,), jnp.int32),
            pltpu.VMEM((BATCH + 16,), jnp.int32),
            pltpu.VMEM((N_INP, 128), jnp.float32),
            pltpu.VMEM((N_INP, 128), jnp.float32),
            pltpu.VMEM((N_INP, 128), jnp.float32),
            pltpu.VMEM((16, N_INP), jnp.float32),
            pltpu.SemaphoreType.DMA,
            pltpu.SemaphoreType.DMA,
            pltpu.SemaphoreType.DMA,
            pltpu.SemaphoreType.DMA,
        ],
        compiler_params=pltpu.CompilerParams(needs_layout_passes=False),
    )(_gather_body)
    return run(tab_u, tab_i, nid_user, nid_item)


# speedup vs baseline: 1.8555x; 1.6072x over previous
import functools

import jax
import jax.numpy as jnp
from jax import lax
from jax.experimental import pallas as pl
from jax.experimental.pallas import tpu as pltpu
from jax.experimental.pallas import tpu_sc as plsc

N_USER = 1000000
N_ITEM = 100000
N_INP = 64
BATCH = 16384

_info = plsc.get_sparse_core_info()
_NC, _NS = _info.num_cores, _info.num_subcores
_NW = _NC * _NS
_BPW = BATCH // _NW
_C = 256
_CI = 128


def _gather_body(user3, item3, nid_u_hbm, nid_i_hbm,
                 out_u_hbm, out_i_hbm,
                 idx_u, idx_i,
                 rows_u0, rows_u1, rows_i0, rows_i1,
                 sem_u0, sem_u1, sem_i0, sem_i1):
    wid = lax.axis_index("s") * _NC + lax.axis_index("c")
    base = wid * _BPW
    pltpu.sync_copy(nid_u_hbm.at[pl.ds(base, _BPW)], idx_u)
    pltpu.sync_copy(nid_i_hbm.at[pl.ds(base, _BPW)], idx_i)

    def burst(tab, idx, rows, sem, off, cnt):
        def issue(k, c):
            vec = idx[pl.ds(off + k * 16, 16)]
            for i in range(16):
                v = vec[i]
                pltpu.make_async_copy(tab.at[v >> 3, v & 7],
                                      rows.at[k * 16 + i], sem).start()
            return c
        lax.fori_loop(0, cnt // 16, issue, 0)

    def drain_write(rows, sem, out, off, cnt):
        pltpu.make_async_copy(out.at[pl.ds(base + off, cnt)], rows, sem).wait()
        pltpu.sync_copy(rows, out.at[pl.ds(base + off, cnt)])

    burst(user3, idx_u, rows_u0, sem_u0, 0, _C)
    burst(user3, idx_u, rows_u1, sem_u1, _C, _C)
    burst(item3, idx_i, rows_i0, sem_i0, 0, _CI)
    burst(item3, idx_i, rows_i1, sem_i1, _CI, _CI)
    drain_write(rows_i0, sem_i0, out_i_hbm, 0, _CI)
    burst(item3, idx_i, rows_i0, sem_i0, 2 * _CI, _CI)
    drain_write(rows_i1, sem_i1, out_i_hbm, _CI, _CI)
    burst(item3, idx_i, rows_i1, sem_i1, 3 * _CI, _CI)
    drain_write(rows_u0, sem_u0, out_u_hbm, 0, _C)
    drain_write(rows_u1, sem_u1, out_u_hbm, _C, _C)
    drain_write(rows_i0, sem_i0, out_i_hbm, 2 * _CI, _CI)
    drain_write(rows_i1, sem_i1, out_i_hbm, 3 * _CI, _CI)


@jax.jit
def kernel(embed_user, embed_item, nid_user, nid_item):
    user3 = embed_user.reshape(N_USER // 8, 8, N_INP)
    item3 = embed_item.reshape(N_ITEM // 8, 8, N_INP)
    mesh = plsc.VectorSubcoreMesh(core_axis_name="c", subcore_axis_name="s")
    run = functools.partial(
        pl.kernel,
        mesh=mesh,
        out_type=(
            jax.ShapeDtypeStruct((BATCH, N_INP), jnp.float32),
            jax.ShapeDtypeStruct((BATCH, N_INP), jnp.float32),
        ),
        scratch_types=[
            pltpu.VMEM((_BPW,), jnp.int32),
            pltpu.VMEM((_BPW,), jnp.int32),
            pltpu.VMEM((_C, N_INP), jnp.float32),
            pltpu.VMEM((_C, N_INP), jnp.float32),
            pltpu.VMEM((_CI, N_INP), jnp.float32),
            pltpu.VMEM((_CI, N_INP), jnp.float32),
            pltpu.SemaphoreType.DMA,
            pltpu.SemaphoreType.DMA,
            pltpu.SemaphoreType.DMA,
            pltpu.SemaphoreType.DMA,
        ],
        compiler_params=pltpu.CompilerParams(needs_layout_passes=False),
    )(_gather_body)
    return run(user3, item3, nid_user, nid_item)


# speedup vs baseline: 1.8581x; 1.0014x over previous
import functools

import jax
import jax.numpy as jnp
from jax import lax
from jax.experimental import pallas as pl
from jax.experimental.pallas import tpu as pltpu
from jax.experimental.pallas import tpu_sc as plsc

N_USER = 1000000
N_ITEM = 100000
N_INP = 64
BATCH = 16384

_info = plsc.get_sparse_core_info()
_NC, _NS = _info.num_cores, _info.num_subcores
_NW = _NC * _NS
_BPW = BATCH // _NW
_C = 256


def _gather_body(tab3, nid_hbm, out_hbm, idx_v, rows0, rows1, sem0, sem1):
    wid = lax.axis_index("s") * _NC + lax.axis_index("c")
    base = wid * _BPW
    pltpu.sync_copy(nid_hbm.at[pl.ds(base, _BPW)], idx_v)

    def burst(rows, sem, off):
        def issue(k, c):
            vec = idx_v[pl.ds(off + k * 16, 16)]
            for i in range(16):
                v = vec[i]
                pltpu.make_async_copy(tab3.at[v >> 3, v & 7],
                                      rows.at[k * 16 + i], sem).start()
            return c
        lax.fori_loop(0, _C // 16, issue, 0)

    def drain_write(rows, sem, off):
        pltpu.make_async_copy(out_hbm.at[pl.ds(base + off, _C)], rows,
                              sem).wait()
        pltpu.sync_copy(rows, out_hbm.at[pl.ds(base + off, _C)])

    burst(rows0, sem0, 0)
    burst(rows1, sem1, _C)
    drain_write(rows0, sem0, 0)
    drain_write(rows1, sem1, _C)


def _make_gather():
    mesh = plsc.VectorSubcoreMesh(core_axis_name="c", subcore_axis_name="s")
    return functools.partial(
        pl.kernel,
        mesh=mesh,
        out_type=jax.ShapeDtypeStruct((BATCH, N_INP), jnp.float32),
        scratch_types=[
            pltpu.VMEM((_BPW,), jnp.int32),
            pltpu.VMEM((_C, N_INP), jnp.float32),
            pltpu.VMEM((_C, N_INP), jnp.float32),
            pltpu.SemaphoreType.DMA,
            pltpu.SemaphoreType.DMA,
        ],
        compiler_params=pltpu.CompilerParams(needs_layout_passes=False),
    )(_gather_body)


@jax.jit
def kernel(embed_user, embed_item, nid_user, nid_item):
    user3 = embed_user.reshape(N_USER // 8, 8, N_INP)
    item3 = embed_item.reshape(N_ITEM // 8, 8, N_INP)
    out_i = _make_gather()(item3, nid_item)
    out_u = _make_gather()(user3, nid_user)
    return (out_u, out_i)
